# baseline (speedup 1.0000x reference)
import functools

import jax
import jax.numpy as jnp
from jax import lax
from jax.experimental import pallas as pl
from jax.experimental.pallas import tpu as pltpu
from jax.experimental.pallas import tpu_sc as plsc

_WIN = 256
_NBUF = 4


def _make_pool(B, L, D):
    info = plsc.get_sparse_core_info()
    NC, NS, LN = info.num_cores, info.num_subcores, info.num_lanes
    NW = NC * NS
    bpw = B // NW
    rpw = bpw * L
    CH, NBUF = _WIN, _NBUF
    nstr = CH // LN
    nch = rpw // CH
    ngrp = nch // NBUF
    rpe = CH // bpw
    mesh = plsc.VectorSubcoreMesh(core_axis_name="c", subcore_axis_name="s")

    @functools.partial(
        pl.kernel,
        mesh=mesh,
        compiler_params=pltpu.CompilerParams(use_tc_tiling_on_sc=False),
        out_type=jax.ShapeDtypeStruct((B, D), jnp.float32),
        scratch_types=[
            pltpu.VMEM((nch, 1, CH), jnp.int32),
            pltpu.VMEM((NBUF, CH, D // 2), jnp.int32),
            pltpu.VMEM((bpw, D), jnp.float32),
            pltpu.SemaphoreType.DMA((NBUF,)),
            pltpu.SemaphoreType.DMA,
        ],
    )
    def pool(x_hbm, emb_hbm, out_hbm, idx_v, rows_v, acc_v, gsem, csem):
        cid = lax.axis_index("c")
        sid = lax.axis_index("s")
        wid = sid * NC + cid

        cp0 = pltpu.async_copy(x_hbm.at[wid], idx_v, csem)

        zero = jnp.zeros((LN,), jnp.float32)

        def zbody(r, carry):
            for k in range(D // LN):
                acc_v[r, pl.ds(LN * k, LN)] = zero
            return carry

        lax.fori_loop(0, bpw, zbody, 0)
        cp0.wait()

        def issue_window(c, b):
            for j in range(nstr):
                iv = idx_v[c, 0, pl.ds(LN * j, LN)]
                pltpu.async_copy(
                    emb_hbm.at[iv], rows_v.at[b, pl.ds(LN * j, LN)],
                    gsem.at[b])

        def drain_window(c, b):
            pltpu.make_async_copy(
                emb_hbm.at[idx_v.at[c, 0]], rows_v.at[b], gsem.at[b]).wait()

        def accum_window(b):
            def abody(r, carry):
                for k in range(D // (2 * LN)):
                    a_e = acc_v[r, pl.ds(2 * LN * k, LN)]
                    a_o = acc_v[r, pl.ds(2 * LN * k + LN, LN)]
                    for t in range(rpe):
                        w32 = rows_v[b, t * bpw + r, pl.ds(LN * k, LN)]
                        e = lax.bitcast_convert_type(
                            lax.shift_left(w32, jnp.int32(16)), jnp.float32)
                        o = lax.bitcast_convert_type(
                            w32 & jnp.int32(-65536), jnp.float32)
                        a_e = a_e + e
                        a_o = a_o + o
                    acc_v[r, pl.ds(2 * LN * k, LN)] = a_e
                    acc_v[r, pl.ds(2 * LN * k + LN, LN)] = a_o
                return carry

            lax.fori_loop(0, bpw, abody, 0)

        for b in range(NBUF):
            issue_window(b, b)

        def grp(g, carry):
            c0 = g * NBUF
            for b in range(NBUF):
                drain_window(c0 + b, b)
                accum_window(b)

                @pl.when(g < ngrp - 1)
                def _():
                    issue_window(c0 + NBUF + b, b)
            return carry

        lax.fori_loop(0, ngrp, grp, 0)
        pltpu.sync_copy(acc_v, out_hbm.at[pl.ds(wid * bpw, bpw)])

    return pool


def _head(pooled, W, b2, scale):
    B, D = pooled.shape
    OUT = W.shape[1]

    def body(p_ref, w_ref, b_ref, o_ref):
        h = jnp.maximum(p_ref[...] * scale, 0.0)
        o_ref[...] = lax.dot_general(
            h, w_ref[...], (((1,), (0,)), ((), ())),
            preferred_element_type=jnp.float32) + b_ref[...]

    return pl.pallas_call(
        body,
        out_shape=jax.ShapeDtypeStruct((B, OUT), jnp.float32),
    )(pooled, W, b2)


def kernel(x, emb, W, b):
    B, L = x.shape
    V, D = emb.shape
    info = plsc.get_sparse_core_info()
    NC, NS = info.num_cores, info.num_subcores
    NW = NC * NS
    bpw = B // NW
    nch = bpw * L // _WIN

    embb = lax.bitcast_convert_type(
        emb.astype(jnp.bfloat16).reshape(V, D // 2, 2), jnp.int32)

    xt = (x.astype(jnp.int32).reshape(NW, bpw, L)
          .transpose(0, 2, 1).reshape(NW, nch, 1, _WIN))

    pooled = _make_pool(B, L, D)(xt, embb)

    perm = []
    for k in range(D // 32):
        perm += [32 * k + 2 * i for i in range(16)]
        perm += [32 * k + 2 * i + 1 for i in range(16)]
    W_perm = W[jnp.array(perm, dtype=jnp.int32)]

    return _head(pooled, W_perm, b.reshape(1, -1), 1.0 / L)


# speedup vs baseline: 2.5856x; 2.5856x over previous
import functools

import jax
import jax.numpy as jnp
from jax import lax
from jax.experimental import pallas as pl
from jax.experimental.pallas import tpu as pltpu
from jax.experimental.pallas import tpu_sc as plsc

_WIN = 256
_NBUF = 4


def _make_pool(B, L, D):
    info = plsc.get_sparse_core_info()
    NC, NS, LN = info.num_cores, info.num_subcores, info.num_lanes
    NW = NC * NS
    bpw = B // NW
    rpw = bpw * L
    CH, NBUF = _WIN, _NBUF
    nstr = CH // LN
    nch = rpw // CH
    ngrp = nch // NBUF
    nvec = D // LN
    mesh = plsc.VectorSubcoreMesh(core_axis_name="c", subcore_axis_name="s")

    @functools.partial(
        pl.kernel,
        mesh=mesh,
        compiler_params=pltpu.CompilerParams(use_tc_tiling_on_sc=False),
        out_type=jax.ShapeDtypeStruct((B, D), jnp.float32),
        scratch_types=[
            pltpu.VMEM((nch, 1, CH), jnp.int32),
            pltpu.VMEM((1, 1, CH), jnp.int32),
            pltpu.VMEM((NBUF, CH, D), jnp.float32),
            pltpu.VMEM((bpw, D), jnp.float32),
            pltpu.VMEM_SHARED((NS * bpw, D), jnp.float32),
            pltpu.SemaphoreType.DMA((NBUF,)),
            pltpu.SemaphoreType.DMA((NBUF,)),
            pltpu.SemaphoreType.DMA,
        ],
    )
    def pool(x_hbm, dst_hbm, emb_hbm, out_hbm,
             idx_v, dst_v, rows_v, zero_v, acc, gsem, ssem, csem):
        cid = lax.axis_index("c")
        sid = lax.axis_index("s")
        wid = sid * NC + cid

        cp0 = pltpu.async_copy(x_hbm.at[wid], idx_v, csem)
        cp1 = pltpu.async_copy(dst_hbm.at[sid], dst_v, csem)

        zero = jnp.zeros((LN,), jnp.float32)

        def zbody(r, carry):
            for k in range(nvec):
                zero_v[r, pl.ds(LN * k, LN)] = zero
            return carry

        lax.fori_loop(0, bpw, zbody, 0)
        pltpu.sync_copy(zero_v, acc.at[pl.ds(sid * bpw, bpw)])
        cp0.wait()
        cp1.wait()

        def issue_window(c, b):
            for j in range(nstr):
                iv = idx_v[c, 0, pl.ds(LN * j, LN)]
                pltpu.async_copy(
                    emb_hbm.at[iv], rows_v.at[b, pl.ds(LN * j, LN)],
                    gsem.at[b])

        def drain_window(c, b):
            pltpu.make_async_copy(
                emb_hbm.at[idx_v.at[c, 0]], rows_v.at[b], gsem.at[b]).wait()

        for b in range(NBUF):
            issue_window(b, b)

        def grp(g, carry):
            c0 = g * NBUF
            cps = []
            for b in range(NBUF):
                drain_window(c0 + b, b)
                cps.append(pltpu.async_copy(
                    rows_v.at[b], acc.at[dst_v.at[0, 0]], ssem.at[b],
                    add=True))
            for b in range(NBUF):
                cps[b].wait()

                @pl.when(g < ngrp - 1)
                def _():
                    issue_window(c0 + NBUF + b, b)
            return carry

        lax.fori_loop(0, ngrp, grp, 0)
        pltpu.sync_copy(acc.at[pl.ds(sid * bpw, bpw)],
                        out_hbm.at[pl.ds(wid * bpw, bpw)])

    return pool


def _head(pooled, W, b2, scale):
    B, D = pooled.shape
    OUT = W.shape[1]

    def body(p_ref, w_ref, b_ref, o_ref):
        h = jnp.maximum(p_ref[...] * scale, 0.0)
        o_ref[...] = lax.dot_general(
            h, w_ref[...], (((1,), (0,)), ((), ())),
            preferred_element_type=jnp.float32) + b_ref[...]

    return pl.pallas_call(
        body,
        out_shape=jax.ShapeDtypeStruct((B, OUT), jnp.float32),
    )(pooled, W, b2)


def kernel(x, emb, W, b):
    B, L = x.shape
    D = emb.shape[1]
    info = plsc.get_sparse_core_info()
    NC, NS = info.num_cores, info.num_subcores
    NW = NC * NS
    bpw = B // NW
    rep = _WIN // bpw
    nch = bpw * L // _WIN

    x32 = (x.astype(jnp.int32).reshape(NW, bpw, L)
           .transpose(0, 2, 1).reshape(NW, nch, 1, _WIN))
    local = jnp.tile(jnp.arange(bpw, dtype=jnp.int32), rep)[None, None, :]
    dst = local + (jnp.arange(NS, dtype=jnp.int32) * bpw)[:, None, None, None]

    pooled = _make_pool(B, L, D)(x32, dst, emb)
    return _head(pooled, W, b.reshape(1, -1), 1.0 / L)
